# Initial kernel scaffold; baseline (speedup 1.0000x reference)
import jax
import jax.numpy as jnp
from jax.experimental import pallas as pl


def kernel(tokens, cu_seqlens, emb_table, W):
    raise NotImplementedError("write your pallas kernel here")


# speedup vs baseline: 3.4410x; 3.4410x over previous
import functools

import jax
import jax.numpy as jnp
from jax import lax
from jax.experimental import pallas as pl
from jax.experimental.pallas import tpu as pltpu
from jax.experimental.pallas import tpu_sc as plsc


def _sc_gather(tokens_2d, emb_table, n_chunks, chunk):
    info = plsc.get_sparse_core_info()
    nw = info.num_cores * info.num_subcores
    t = nw * n_chunks * chunk
    d = emb_table.shape[1]
    per_w = n_chunks * chunk
    mesh = plsc.VectorSubcoreMesh(core_axis_name="c", subcore_axis_name="s")

    @functools.partial(
        pl.kernel,
        mesh=mesh,
        out_type=jax.ShapeDtypeStruct((t, d), jnp.float32),
        scratch_types=[
            pltpu.VMEM((n_chunks, chunk), jnp.int32),
            pltpu.VMEM((chunk, d), jnp.float32),
            pltpu.VMEM((chunk, d), jnp.float32),
            pltpu.SemaphoreType.DMA,
            pltpu.SemaphoreType.DMA,
        ],
    )
    def gather_kernel(tok_hbm, table_hbm, out_hbm, idx_v, rows0, rows1, sem0, sem1):
        wid = lax.axis_index("s") * info.num_cores + lax.axis_index("c")
        base = wid * per_w
        pltpu.sync_copy(tok_hbm.at[wid], idx_v)
        bufs = (rows0, rows1)
        sems = (sem0, sem1)
        cps = [None, None]
        cps[0] = pltpu.async_copy(table_hbm.at[idx_v.at[0]], bufs[0], sems[0])
        for c in range(n_chunks):
            nxt = c + 1
            if nxt < n_chunks:
                cps[nxt % 2] = pltpu.async_copy(
                    table_hbm.at[idx_v.at[nxt]], bufs[nxt % 2], sems[nxt % 2]
                )
            cps[c % 2].wait()
            pltpu.sync_copy(bufs[c % 2], out_hbm.at[pl.ds(base + c * chunk, chunk)])

    return gather_kernel(tokens_2d, emb_table)


def _tc_compute(emb, cu_seqlens, W, block_t):
    t, d = emb.shape
    b = cu_seqlens.shape[0] - 1
    grid = t // block_t

    def body(cu_ref, emb_ref, w_ref, out_ref, acc_ref):
        g = pl.program_id(0)

        @pl.when(g == 0)
        def _init():
            acc_ref[...] = jnp.zeros_like(acc_ref)

        e = emb_ref[...]
        gate = jax.nn.sigmoid(jnp.dot(e, w_ref[...], preferred_element_type=jnp.float32))
        h = e * gate

        pos = g * block_t + lax.broadcasted_iota(jnp.int32, (1, block_t), 1)
        seg = jnp.zeros((1, block_t), jnp.int32)
        for j in range(1, b):
            seg = seg + (pos >= cu_ref[j]).astype(jnp.int32)
        onehot = (lax.broadcasted_iota(jnp.int32, (b, block_t), 0) == seg).astype(
            jnp.float32
        )
        acc_ref[...] += jnp.dot(onehot, h, preferred_element_type=jnp.float32)

        @pl.when(g == grid - 1)
        def _fin():
            rid = lax.broadcasted_iota(jnp.int32, (b, 1), 0)
            lens = jnp.zeros((b, 1), jnp.float32)
            for j in range(b):
                lens = lens + jnp.where(
                    rid == j, (cu_ref[j + 1] - cu_ref[j]).astype(jnp.float32), 0.0
                )
            out_ref[...] = acc_ref[...] / jnp.maximum(lens, 1.0)

    return pl.pallas_call(
        body,
        grid_spec=pltpu.PrefetchScalarGridSpec(
            num_scalar_prefetch=1,
            grid=(grid,),
            in_specs=[
                pl.BlockSpec((block_t, d), lambda g, cu: (g, 0)),
                pl.BlockSpec((d, d), lambda g, cu: (0, 0)),
            ],
            out_specs=pl.BlockSpec((b, d), lambda g, cu: (0, 0)),
            scratch_shapes=[pltpu.VMEM((b, d), jnp.float32)],
        ),
        out_shape=jax.ShapeDtypeStruct((b, d), jnp.float32),
    )(cu_seqlens, emb, W)


def kernel(tokens, cu_seqlens, emb_table, W):
    t = tokens.shape[0]
    info = plsc.get_sparse_core_info()
    nw = info.num_cores * info.num_subcores
    chunk = 128
    n_chunks = t // (nw * chunk)
    tokens_2d = tokens.reshape(nw, n_chunks, chunk)
    emb = _sc_gather(tokens_2d, emb_table, n_chunks, chunk)
    return _tc_compute(emb, cu_seqlens, W, block_t=512)


# speedup vs baseline: 4.2439x; 1.2333x over previous
import functools

import jax
import jax.numpy as jnp
from jax import lax
from jax.experimental import pallas as pl
from jax.experimental.pallas import tpu as pltpu
from jax.experimental.pallas import tpu_sc as plsc


def _sc_gather(tokens_2d, emb_table, n_chunks, chunk):
    info = plsc.get_sparse_core_info()
    nw = info.num_cores * info.num_subcores
    t = nw * n_chunks * chunk
    d = emb_table.shape[1]
    per_w = n_chunks * chunk
    mesh = plsc.VectorSubcoreMesh(core_axis_name="c", subcore_axis_name="s")

    @functools.partial(
        pl.kernel,
        mesh=mesh,
        out_type=jax.ShapeDtypeStruct((t, d), jnp.float32),
        scratch_types=[
            pltpu.VMEM((n_chunks, chunk), jnp.int32),
            pltpu.VMEM((chunk, d), jnp.float32),
            pltpu.VMEM((chunk, d), jnp.float32),
            pltpu.SemaphoreType.DMA,
            pltpu.SemaphoreType.DMA,
        ],
    )
    def gather_kernel(tok_hbm, table_hbm, out_hbm, idx_v, rows0, rows1, sem0, sem1):
        wid = lax.axis_index("s") * info.num_cores + lax.axis_index("c")
        base = wid * per_w
        pltpu.sync_copy(tok_hbm.at[wid], idx_v)
        bufs = (rows0, rows1)
        sems = (sem0, sem1)
        cps = [None, None]
        cps[0] = pltpu.async_copy(table_hbm.at[idx_v.at[0]], bufs[0], sems[0])
        for c in range(n_chunks):
            nxt = c + 1
            if nxt < n_chunks:
                cps[nxt % 2] = pltpu.async_copy(
                    table_hbm.at[idx_v.at[nxt]], bufs[nxt % 2], sems[nxt % 2]
                )
            cps[c % 2].wait()
            pltpu.sync_copy(bufs[c % 2], out_hbm.at[pl.ds(base + c * chunk, chunk)])

    return gather_kernel(tokens_2d, emb_table)


def _tc_compute(emb, cu_seqlens, W, block_t):
    t, d = emb.shape
    b = cu_seqlens.shape[0] - 1
    grid = t // block_t

    def body(cu_ref, emb_ref, w_ref, out_ref, acc_ref):
        g = pl.program_id(0)

        @pl.when(g == 0)
        def _init():
            acc_ref[...] = jnp.zeros_like(acc_ref)

        e = emb_ref[...]
        gate = jax.nn.sigmoid(
            jnp.dot(
                e.astype(jnp.bfloat16),
                w_ref[...].astype(jnp.bfloat16),
                preferred_element_type=jnp.float32,
            )
        )
        h = e * gate

        pos = g * block_t + lax.broadcasted_iota(jnp.int32, (1, block_t), 1)
        seg = jnp.zeros((1, block_t), jnp.int32)
        for j in range(1, b):
            seg = seg + (pos >= cu_ref[j]).astype(jnp.int32)
        onehot = (lax.broadcasted_iota(jnp.int32, (b, block_t), 0) == seg).astype(
            jnp.bfloat16
        )
        acc_ref[...] += jnp.dot(
            onehot, h.astype(jnp.bfloat16), preferred_element_type=jnp.float32
        )

        @pl.when(g == grid - 1)
        def _fin():
            rid = lax.broadcasted_iota(jnp.int32, (b, 1), 0)
            lens = jnp.zeros((b, 1), jnp.float32)
            for j in range(b):
                lens = lens + jnp.where(
                    rid == j, (cu_ref[j + 1] - cu_ref[j]).astype(jnp.float32), 0.0
                )
            out_ref[...] = acc_ref[...] / jnp.maximum(lens, 1.0)

    return pl.pallas_call(
        body,
        grid_spec=pltpu.PrefetchScalarGridSpec(
            num_scalar_prefetch=1,
            grid=(grid,),
            in_specs=[
                pl.BlockSpec((block_t, d), lambda g, cu: (g, 0)),
                pl.BlockSpec((d, d), lambda g, cu: (0, 0)),
            ],
            out_specs=pl.BlockSpec((b, d), lambda g, cu: (0, 0)),
            scratch_shapes=[pltpu.VMEM((b, d), jnp.float32)],
        ),
        out_shape=jax.ShapeDtypeStruct((b, d), jnp.float32),
    )(cu_seqlens, emb, W)


def kernel(tokens, cu_seqlens, emb_table, W):
    t = tokens.shape[0]
    info = plsc.get_sparse_core_info()
    nw = info.num_cores * info.num_subcores
    chunk = 128
    n_chunks = t // (nw * chunk)
    tokens_2d = tokens.reshape(nw, n_chunks, chunk)
    emb = _sc_gather(tokens_2d, emb_table, n_chunks, chunk)
    return _tc_compute(emb, cu_seqlens, W, block_t=1024)


# speedup vs baseline: 4.8628x; 1.1458x over previous
import functools

import jax
import jax.numpy as jnp
from jax import lax
from jax.experimental import pallas as pl
from jax.experimental.pallas import tpu as pltpu
from jax.experimental.pallas import tpu_sc as plsc


def _sc_gather(tokens_2d, emb_table, n_chunks, chunk):
    info = plsc.get_sparse_core_info()
    nw = info.num_cores * info.num_subcores
    t = nw * n_chunks * chunk
    d = emb_table.shape[1]
    per_w = n_chunks * chunk
    mesh = plsc.VectorSubcoreMesh(core_axis_name="c", subcore_axis_name="s")

    @functools.partial(
        pl.kernel,
        mesh=mesh,
        out_type=jax.ShapeDtypeStruct((t, d), jnp.float32),
        scratch_types=[
            pltpu.VMEM((n_chunks, chunk), jnp.int32),
            pltpu.VMEM((chunk, d), jnp.float32),
            pltpu.VMEM((chunk, d), jnp.float32),
            pltpu.SemaphoreType.DMA,
            pltpu.SemaphoreType.DMA,
        ],
    )
    def gather_kernel(tok_hbm, table_hbm, out_hbm, idx_v, rows0, rows1, sem0, sem1):
        wid = lax.axis_index("s") * info.num_cores + lax.axis_index("c")
        base = wid * per_w
        pltpu.sync_copy(tok_hbm.at[wid], idx_v)
        bufs = (rows0, rows1)
        sems = (sem0, sem1)
        cps = [None, None]
        cps[0] = pltpu.async_copy(table_hbm.at[idx_v.at[0]], bufs[0], sems[0])
        for c in range(n_chunks):
            nxt = c + 1
            if nxt < n_chunks:
                cps[nxt % 2] = pltpu.async_copy(
                    table_hbm.at[idx_v.at[nxt]], bufs[nxt % 2], sems[nxt % 2]
                )
            cps[c % 2].wait()
            pltpu.sync_copy(bufs[c % 2], out_hbm.at[pl.ds(base + c * chunk, chunk)])

    return gather_kernel(tokens_2d, emb_table)


def _tc_compute(emb, cu_seqlens, W, block_t):
    t, d = emb.shape
    b = cu_seqlens.shape[0] - 1
    grid = t // block_t

    def body(cu_ref, emb_ref, w_ref, out_ref, acc_ref):
        g = pl.program_id(0)

        @pl.when(g == 0)
        def _init():
            acc_ref[...] = jnp.zeros_like(acc_ref)

        e = emb_ref[...]
        gate = jax.nn.sigmoid(
            jnp.dot(
                e.astype(jnp.bfloat16),
                w_ref[...].astype(jnp.bfloat16),
                preferred_element_type=jnp.float32,
            )
        )
        h = e * gate

        pos = g * block_t + lax.broadcasted_iota(jnp.int32, (1, block_t), 1)
        seg = jnp.zeros((1, block_t), jnp.int32)
        for j in range(1, b):
            seg = seg + (pos >= cu_ref[j]).astype(jnp.int32)
        onehot = (lax.broadcasted_iota(jnp.int32, (b, block_t), 0) == seg).astype(
            jnp.bfloat16
        )
        acc_ref[...] += jnp.dot(
            onehot, h.astype(jnp.bfloat16), preferred_element_type=jnp.float32
        )

        @pl.when(g == grid - 1)
        def _fin():
            rid = lax.broadcasted_iota(jnp.int32, (b, 1), 0)
            lens = jnp.zeros((b, 1), jnp.float32)
            for j in range(b):
                lens = lens + jnp.where(
                    rid == j, (cu_ref[j + 1] - cu_ref[j]).astype(jnp.float32), 0.0
                )
            out_ref[...] = acc_ref[...] / jnp.maximum(lens, 1.0)

    return pl.pallas_call(
        body,
        grid_spec=pltpu.PrefetchScalarGridSpec(
            num_scalar_prefetch=1,
            grid=(grid,),
            in_specs=[
                pl.BlockSpec((block_t, d), lambda g, cu: (g, 0)),
                pl.BlockSpec((d, d), lambda g, cu: (0, 0)),
            ],
            out_specs=pl.BlockSpec((b, d), lambda g, cu: (0, 0)),
            scratch_shapes=[pltpu.VMEM((b, d), jnp.float32)],
        ),
        out_shape=jax.ShapeDtypeStruct((b, d), jnp.float32),
    )(cu_seqlens, emb, W)


def kernel(tokens, cu_seqlens, emb_table, W):
    t = tokens.shape[0]
    info = plsc.get_sparse_core_info()
    nw = info.num_cores * info.num_subcores
    chunk = 128
    n_chunks = t // (nw * chunk)
    tokens_2d = tokens.reshape(nw, n_chunks, chunk)
    emb = _sc_gather(tokens_2d, emb_table, n_chunks, chunk)
    return _tc_compute(emb, cu_seqlens, W, block_t=2048)


# speedup vs baseline: 5.2452x; 1.0786x over previous
import functools

import jax
import jax.numpy as jnp
from jax import lax
from jax.experimental import pallas as pl
from jax.experimental.pallas import tpu as pltpu
from jax.experimental.pallas import tpu_sc as plsc


def _sc_gather(tokens_2d, emb_table, n_chunks, chunk):
    info = plsc.get_sparse_core_info()
    nw = info.num_cores * info.num_subcores
    t = nw * n_chunks * chunk
    d = emb_table.shape[1]
    per_w = n_chunks * chunk
    mesh = plsc.VectorSubcoreMesh(core_axis_name="c", subcore_axis_name="s")

    @functools.partial(
        pl.kernel,
        mesh=mesh,
        out_type=jax.ShapeDtypeStruct((t, d), jnp.float32),
        scratch_types=[
            pltpu.VMEM((n_chunks, chunk), jnp.int32),
            pltpu.VMEM((chunk, d), jnp.float32),
            pltpu.VMEM((chunk, d), jnp.float32),
            pltpu.SemaphoreType.DMA,
            pltpu.SemaphoreType.DMA,
        ],
    )
    def gather_kernel(tok_hbm, table_hbm, out_hbm, idx_v, rows0, rows1, sem0, sem1):
        wid = lax.axis_index("s") * info.num_cores + lax.axis_index("c")
        base = wid * per_w
        pltpu.sync_copy(tok_hbm.at[wid], idx_v)
        bufs = (rows0, rows1)
        sems = (sem0, sem1)
        cps = [None, None]
        cps[0] = pltpu.async_copy(table_hbm.at[idx_v.at[0]], bufs[0], sems[0])
        for c in range(n_chunks):
            nxt = c + 1
            if nxt < n_chunks:
                cps[nxt % 2] = pltpu.async_copy(
                    table_hbm.at[idx_v.at[nxt]], bufs[nxt % 2], sems[nxt % 2]
                )
            cps[c % 2].wait()
            pltpu.sync_copy(bufs[c % 2], out_hbm.at[pl.ds(base + c * chunk, chunk)])

    return gather_kernel(tokens_2d, emb_table)


def _tc_compute(emb, cu_seqlens, W, block_t):
    t, d = emb.shape
    b = cu_seqlens.shape[0] - 1
    grid = t // block_t

    def body(cu_ref, emb_ref, w_ref, out_ref, acc_ref):
        g = pl.program_id(0)

        @pl.when(g == 0)
        def _init():
            acc_ref[...] = jnp.zeros_like(acc_ref)

        e = emb_ref[...]
        logits = jnp.dot(
            e.astype(jnp.bfloat16),
            w_ref[...].astype(jnp.bfloat16),
            preferred_element_type=jnp.float32,
        )
        gate = 0.5 * jnp.tanh(0.5 * logits) + 0.5
        h = e * gate

        pos = g * block_t + lax.broadcasted_iota(jnp.int32, (1, block_t), 1)
        seg = jnp.zeros((1, block_t), jnp.int32)
        for j in range(1, b):
            seg = seg + (pos >= cu_ref[j]).astype(jnp.int32)
        onehot = (lax.broadcasted_iota(jnp.int32, (b, block_t), 0) == seg).astype(
            jnp.bfloat16
        )
        acc_ref[...] += jnp.dot(
            onehot, h.astype(jnp.bfloat16), preferred_element_type=jnp.float32
        )

        @pl.when(g == grid - 1)
        def _fin():
            rid = lax.broadcasted_iota(jnp.int32, (b, 1), 0)
            lens = jnp.zeros((b, 1), jnp.float32)
            for j in range(b):
                lens = lens + jnp.where(
                    rid == j, (cu_ref[j + 1] - cu_ref[j]).astype(jnp.float32), 0.0
                )
            out_ref[...] = acc_ref[...] / jnp.maximum(lens, 1.0)

    return pl.pallas_call(
        body,
        grid_spec=pltpu.PrefetchScalarGridSpec(
            num_scalar_prefetch=1,
            grid=(grid,),
            in_specs=[
                pl.BlockSpec((block_t, d), lambda g, cu: (g, 0)),
                pl.BlockSpec((d, d), lambda g, cu: (0, 0)),
            ],
            out_specs=pl.BlockSpec((b, d), lambda g, cu: (0, 0)),
            scratch_shapes=[pltpu.VMEM((b, d), jnp.float32)],
        ),
        out_shape=jax.ShapeDtypeStruct((b, d), jnp.float32),
    )(cu_seqlens, emb, W)


def kernel(tokens, cu_seqlens, emb_table, W):
    t = tokens.shape[0]
    info = plsc.get_sparse_core_info()
    nw = info.num_cores * info.num_subcores
    chunk = 128
    n_chunks = t // (nw * chunk)
    tokens_2d = tokens.reshape(nw, n_chunks, chunk)
    emb = _sc_gather(tokens_2d, emb_table, n_chunks, chunk)
    return _tc_compute(emb, cu_seqlens, W, block_t=4096)
